# baseline (speedup 1.0000x reference)
import functools

import jax
import jax.numpy as jnp
from jax import lax
from jax.experimental import pallas as pl
from jax.experimental.pallas import tpu as pltpu
from jax.experimental.pallas import tpu_sc as plsc

T = 2048
H = 1024
E = 16
I = 512
IS = 1024
N_GROUP = 4
GROUP_SIZE = E // N_GROUP
SCALE = 2.5

_BT = 256
_NB = T // _BT
_NA = 2 * T
_BR = 128
_NWI = _NA // _BR + E
_NAP = (_NA // _BR + E + 1) * _BR
_PAD_TILE = _NAP // _BR - 1
_NEG = -1e30

_NC = 2
_NS = 16
_NW = _NC * _NS
_TPW = T // _NW
_CH = 16
_HF = 32


def _router_body(x_ref, rwt_ref, bias_ref,
                 wr0_ref, wr1_ref, e1_ref, e2_ref,
                 p1_ref, p2_ref, cnt_ref, counts_vmem, tri_vmem):
    blk = pl.program_id(0)
    x = x_ref[...]
    logits = jnp.dot(x, rwt_ref[...], preferred_element_type=jnp.float32)
    scores = jax.nn.sigmoid(logits)
    sfc = scores + bias_ref[...]

    cols = lax.broadcasted_iota(jnp.int32, (_BT, E), 1)
    grp = cols // GROUP_SIZE

    gscores = []
    for g in range(N_GROUP):
        vals = jnp.where(grp == g, sfc, _NEG)
        m1 = jnp.max(vals, axis=1, keepdims=True)
        i1 = jnp.min(jnp.where(vals == m1, cols, E + 1), axis=1, keepdims=True)
        vals2 = jnp.where(cols == i1, _NEG, vals)
        m2 = jnp.max(vals2, axis=1, keepdims=True)
        gscores.append(m1 + m2)

    gm1 = gscores[0]
    gi1 = jnp.zeros_like(gm1, dtype=jnp.int32)
    for g in range(1, N_GROUP):
        better = gscores[g] > gm1
        gi1 = jnp.where(better, g, gi1)
        gm1 = jnp.maximum(gscores[g], gm1)
    gm2 = jnp.full_like(gm1, _NEG)
    gi2 = jnp.zeros_like(gi1)
    for g in range(N_GROUP):
        cand = jnp.where(gi1 == g, _NEG, gscores[g])
        better = cand > gm2
        gi2 = jnp.where(better, g, gi2)
        gm2 = jnp.maximum(cand, gm2)

    group_mask = (grp == gi1) | (grp == gi2)
    msfc = jnp.where(group_mask, sfc, 0.0)

    m1 = jnp.max(msfc, axis=1, keepdims=True)
    e1 = jnp.min(jnp.where(msfc == m1, cols, E + 1), axis=1, keepdims=True)
    msfc2 = jnp.where(cols == e1, _NEG, msfc)
    m2 = jnp.max(msfc2, axis=1, keepdims=True)
    e2 = jnp.min(jnp.where(msfc2 == m2, cols, E + 1), axis=1, keepdims=True)

    sel1 = cols == e1
    sel2 = cols == e2
    w1 = jnp.sum(jnp.where(sel1, scores, 0.0), axis=1, keepdims=True)
    w2 = jnp.sum(jnp.where(sel2, scores, 0.0), axis=1, keepdims=True)
    denom = w1 + w2 + 1e-20
    w1 = w1 / denom * SCALE
    w2 = w2 / denom * SCALE

    wr0_ref[...] = jnp.broadcast_to(w1, (_BT, E))
    wr1_ref[...] = jnp.broadcast_to(w2, (_BT, E))
    e1_ref[...] = e1
    e2_ref[...] = e2

    onehot = jnp.concatenate([sel1, sel2], axis=0).astype(jnp.float32)

    @pl.when(blk == 0)
    def _():
        ri = lax.broadcasted_iota(jnp.int32, (2 * _BT, 2 * _BT), 0)
        ci = lax.broadcasted_iota(jnp.int32, (2 * _BT, 2 * _BT), 1)
        tri_vmem[...] = (ci < ri).astype(jnp.float32)

    prank = jnp.dot(tri_vmem[...], onehot,
                    preferred_element_type=jnp.float32)
    base = jnp.where(blk == 0, 0.0, counts_vmem[0:1, 0:E])
    prank = prank + base
    p1 = jnp.sum(jnp.where(sel1, prank[:_BT], 0.0), axis=1, keepdims=True)
    p2 = jnp.sum(jnp.where(sel2, prank[_BT:], 0.0), axis=1, keepdims=True)
    p1_ref[...] = p1.astype(jnp.int32)
    p2_ref[...] = p2.astype(jnp.int32)
    new_counts = base + jnp.sum(onehot, axis=0, keepdims=True)
    counts_vmem[0:1, 0:E] = new_counts
    cnt_ref[...] = new_counts.astype(jnp.int32)


def _router(hidden_states, router_weight, bias):
    return pl.pallas_call(
        _router_body,
        grid=(_NB,),
        in_specs=[
            pl.BlockSpec((_BT, H), lambda i: (i, 0)),
            pl.BlockSpec((H, E), lambda i: (0, 0)),
            pl.BlockSpec((1, E), lambda i: (0, 0)),
        ],
        out_specs=[
            pl.BlockSpec((_BT, E), lambda i: (i, 0)),
            pl.BlockSpec((_BT, E), lambda i: (i, 0)),
            pl.BlockSpec((_BT, 1), lambda i: (i, 0)),
            pl.BlockSpec((_BT, 1), lambda i: (i, 0)),
            pl.BlockSpec((_BT, 1), lambda i: (i, 0)),
            pl.BlockSpec((_BT, 1), lambda i: (i, 0)),
            pl.BlockSpec((1, E), lambda i: (0, 0)),
        ],
        out_shape=[
            jax.ShapeDtypeStruct((T, E), jnp.float32),
            jax.ShapeDtypeStruct((T, E), jnp.float32),
            jax.ShapeDtypeStruct((T, 1), jnp.int32),
            jax.ShapeDtypeStruct((T, 1), jnp.int32),
            jax.ShapeDtypeStruct((T, 1), jnp.int32),
            jax.ShapeDtypeStruct((T, 1), jnp.int32),
            jax.ShapeDtypeStruct((1, E), jnp.int32),
        ],
        scratch_shapes=[pltpu.VMEM((8, 128), jnp.float32),
                        pltpu.VMEM((2 * _BT, 2 * _BT), jnp.float32)],
        compiler_params=pltpu.CompilerParams(
            dimension_semantics=("arbitrary",)),
    )(hidden_states, router_weight.T, bias.reshape(1, E))


def _shared_body(x_ref, sup_ref, sdn_ref, ysh_ref):
    xb = x_ref[...].astype(jnp.bfloat16)
    h = jnp.dot(xb, sup_ref[...], preferred_element_type=jnp.float32)
    r = jnp.maximum(h, 0.0)
    rr = (r * r).astype(jnp.bfloat16)
    ysh_ref[...] = jnp.dot(rr, sdn_ref[...], preferred_element_type=jnp.float32)


def _shared_mlp(hidden_states, sup, sdn):
    return pl.pallas_call(
        _shared_body,
        grid=(_NB,),
        in_specs=[
            pl.BlockSpec((_BT, H), lambda i: (i, 0)),
            pl.BlockSpec((H, IS), lambda i: (0, 0)),
            pl.BlockSpec((IS, H), lambda i: (0, 0)),
        ],
        out_specs=pl.BlockSpec((_BT, H), lambda i: (i, 0)),
        out_shape=jax.ShapeDtypeStruct((T, H), jnp.float32),
    )(hidden_states, sup, sdn)


def _poff_from_counts(cnt_v):
    pcnt = ((cnt_v + (_BR - 1)) // _BR) * _BR
    return plsc.cumsum(pcnt) - pcnt, pcnt


def _lane_select(vec, e):
    lidx = lax.broadcasted_iota(jnp.int32, (E,), 0)
    return jnp.sum(jnp.where(lidx == e, vec, 0))


def _sc_mesh():
    return plsc.VectorSubcoreMesh(core_axis_name="c", subcore_axis_name="s")


def _dispatch_body(hid, e1a, e2a, p1a, p2a, cnta, xs, wout,
                   rows_v, e1_v, e2_v, p1_v, p2_v, cnt_v, poff_v, wit_v, sem):
    wid = lax.axis_index("s") * _NC + lax.axis_index("c")
    base = wid * _TPW
    pltpu.sync_copy(cnta, cnt_v)
    pltpu.sync_copy(hid.at[pl.ds(base, _TPW)], rows_v)
    pltpu.sync_copy(e1a.at[pl.ds(base, _TPW)], e1_v)
    pltpu.sync_copy(e2a.at[pl.ds(base, _TPW)], e2_v)
    pltpu.sync_copy(p1a.at[pl.ds(base, _TPW)], p1_v)
    pltpu.sync_copy(p2a.at[pl.ds(base, _TPW)], p2_v)

    cnt = cnt_v[...]
    poff, _ = _poff_from_counts(cnt)
    poff_v[...] = poff

    for ch in range(_TPW // _CH):
        sl = pl.ds(ch * _CH, _CH)
        p1_v[sl] = plsc.load_gather(poff_v, [e1_v[sl]]) + p1_v[sl]
        p2_v[sl] = plsc.load_gather(poff_v, [e2_v[sl]]) + p2_v[sl]
    cps = [pltpu.async_copy(rows_v, xs.at[p1_v], sem),
           pltpu.async_copy(rows_v, xs.at[p2_v], sem)]

    ntile = (cnt + (_BR - 1)) // _BR
    tbsum = plsc.cumsum(ntile)
    tb = tbsum - ntile
    total = jnp.max(tbsum)
    for half in range(_NWI // E):
        cvec = lax.broadcasted_iota(jnp.int32, (E,), 0) + half * E
        e_of = jnp.zeros((E,), jnp.int32)
        for e in range(E):
            e_of = e_of + (jnp.where(_lane_select(tb, e) <= cvec, 1, 0))
        e_of = e_of - 1
        jv = cvec - _gather_reg(tb, e_of, poff_v)
        st = _gather_reg(poff, e_of, poff_v) + jv * _BR
        seg_end = (_gather_reg(poff, e_of, poff_v)
                   + _gather_reg(cnt, e_of, poff_v))
        en = jnp.minimum(st + _BR, seg_end)
        msk = cvec < total
        sl = pl.ds(half * E, E)
        wit_v[0, sl] = jnp.where(msk, e_of, 0)
        wit_v[1, sl] = jnp.where(msk, st // _BR, _PAD_TILE)
        wit_v[2, sl] = jnp.where(msk, en, 0)
        wit_v[3, sl] = jnp.zeros((E,), jnp.int32)

    @pl.when(wid == 0)
    def _():
        pltpu.sync_copy(wit_v, wout)

    for cp in cps:
        cp.wait()


def _gather_reg(vec, idx, scratch_v):
    scratch_v[...] = vec
    return plsc.load_gather(scratch_v, [idx])


def _sc_dispatch(hidden_states, e1a, e2a, p1a, p2a, cnta):
    k = functools.partial(
        pl.kernel,
        mesh=_sc_mesh(),
        out_type=[
            jax.ShapeDtypeStruct((_NAP, H), jnp.float32),
            jax.ShapeDtypeStruct((4, _NWI), jnp.int32),
        ],
        scratch_types=[
            pltpu.VMEM((_TPW, H), jnp.float32),
            pltpu.VMEM((_TPW,), jnp.int32),
            pltpu.VMEM((_TPW,), jnp.int32),
            pltpu.VMEM((_TPW,), jnp.int32),
            pltpu.VMEM((_TPW,), jnp.int32),
            pltpu.VMEM((E,), jnp.int32),
            pltpu.VMEM((E,), jnp.int32),
            pltpu.VMEM((4, _NWI), jnp.int32),
            pltpu.SemaphoreType.DMA,
        ],
        compiler_params=pltpu.CompilerParams(needs_layout_passes=False),
    )(_dispatch_body)
    return k(hidden_states, e1a, e2a, p1a, p2a, cnta)


def _gmm_body(wit_ref, xs_ref, up_ref, dn_ref, out_ref):
    wi = pl.program_id(0)
    tile = wit_ref[1, wi]
    end = wit_ref[2, wi]

    @pl.when(end > 0)
    def _():
        x = xs_ref[...].astype(jnp.bfloat16)
        up = up_ref[0].astype(jnp.bfloat16)
        dn = dn_ref[0].astype(jnp.bfloat16)
        h = jnp.dot(x, up, preferred_element_type=jnp.float32)
        r = jnp.maximum(h, 0.0)
        rr = (r * r).astype(jnp.bfloat16)
        yc = jnp.dot(rr, dn, preferred_element_type=jnp.float32)
        rows = tile * _BR + lax.broadcasted_iota(jnp.int32, (_BR, 1), 0)
        out_ref[...] = jnp.where(rows < end, yc, 0.0)


def _gmm(witems, xs, up_w, down_w):
    return pl.pallas_call(
        _gmm_body,
        grid_spec=pltpu.PrefetchScalarGridSpec(
            num_scalar_prefetch=1,
            grid=(_NWI,),
            in_specs=[
                pl.BlockSpec((_BR, H), lambda wi, wa: (wa[1, wi], 0)),
                pl.BlockSpec((1, H, I), lambda wi, wa: (wa[0, wi], 0, 0)),
                pl.BlockSpec((1, I, H), lambda wi, wa: (wa[0, wi], 0, 0)),
            ],
            out_specs=pl.BlockSpec((_BR, H), lambda wi, wa: (wa[1, wi], 0)),
        ),
        out_shape=jax.ShapeDtypeStruct((_NAP, H), jnp.float32),
        compiler_params=pltpu.CompilerParams(
            dimension_semantics=("arbitrary",)),
    )(witems, xs, up_w, down_w)


def _combine_body(ysh, outs, e1a, e2a, p1a, p2a, cnta, wr0, wr1, y,
                  acc_v, r0_v, r1_v, r0b_v, r1b_v,
                  e1_v, e2_v, p1_v, p2_v, cnt_v, poff_v, w0_v, w1_v,
                  i0a_v, i1a_v, i0b_v, i1b_v, sem):
    wid = lax.axis_index("s") * _NC + lax.axis_index("c")
    base = wid * _TPW
    pltpu.sync_copy(cnta, cnt_v)
    pltpu.sync_copy(e1a.at[pl.ds(base, _TPW)], e1_v)
    pltpu.sync_copy(e2a.at[pl.ds(base, _TPW)], e2_v)
    pltpu.sync_copy(p1a.at[pl.ds(base, _TPW)], p1_v)
    pltpu.sync_copy(p2a.at[pl.ds(base, _TPW)], p2_v)
    pltpu.sync_copy(wr0.at[pl.ds(base, _TPW)], w0_v)
    pltpu.sync_copy(wr1.at[pl.ds(base, _TPW)], w1_v)
    poff, _ = _poff_from_counts(cnt_v[...])
    poff_v[...] = poff
    for ch in range(_TPW // _CH):
        sl = pl.ds(ch * _CH, _CH)
        p1_v[sl] = plsc.load_gather(poff_v, [e1_v[sl]]) + p1_v[sl]
        p2_v[sl] = plsc.load_gather(poff_v, [e2_v[sl]]) + p2_v[sl]

    nch = _TPW // _CH
    bufs = [(r0_v, r1_v), (r0b_v, r1b_v)]
    ibufs = [(i0a_v, i1a_v), (i0b_v, i1b_v)]

    def fire(c):
        sl = pl.ds(c * _CH, _CH)
        r0, r1 = bufs[c % 2]
        ia, ib = ibufs[c % 2]
        ia[...] = p1_v[sl]
        ib[...] = p2_v[sl]
        return (pltpu.async_copy(outs.at[ia], r0, sem),
                pltpu.async_copy(outs.at[ib], r1, sem))

    chunks_per_half = _HF // _CH
    cps = fire(0)
    for c in range(nch):
        hf, cl = divmod(c, chunks_per_half)
        if cl == 0:
            pltpu.sync_copy(ysh.at[pl.ds(base + hf * _HF, _HF)], acc_v)
        cps[0].wait()
        cps[1].wait()
        if c + 1 < nch:
            nxt = fire(c + 1)
        r0, r1 = bufs[c % 2]
        for r in range(_CH):
            row = c * _CH + r
            arow = cl * _CH + r
            wv0 = w0_v[row, :]
            wv1 = w1_v[row, :]

            @plsc.parallel_loop(0, H, step=16, unroll=8)
            def _(off, arow=arow, r=r, wv0=wv0, wv1=wv1, r0=r0, r1=r1):
                sl = pl.ds(off, 16)
                acc_v[arow, sl] = (acc_v[arow, sl] + wv0 * r0[r, sl]
                                   + wv1 * r1[r, sl])

        if cl == chunks_per_half - 1:
            pltpu.sync_copy(acc_v, y.at[pl.ds(base + hf * _HF, _HF)])
        if c + 1 < nch:
            cps = nxt


def _sc_combine(ysh, outs, e1a, e2a, p1a, p2a, cnta, wr0, wr1):
    k = functools.partial(
        pl.kernel,
        mesh=_sc_mesh(),
        out_type=jax.ShapeDtypeStruct((T, H), jnp.float32),
        scratch_types=[
            pltpu.VMEM((_HF, H), jnp.float32),
            pltpu.VMEM((_CH, H), jnp.float32),
            pltpu.VMEM((_CH, H), jnp.float32),
            pltpu.VMEM((_CH, H), jnp.float32),
            pltpu.VMEM((_CH, H), jnp.float32),
            pltpu.VMEM((_TPW,), jnp.int32),
            pltpu.VMEM((_TPW,), jnp.int32),
            pltpu.VMEM((_TPW,), jnp.int32),
            pltpu.VMEM((_TPW,), jnp.int32),
            pltpu.VMEM((E,), jnp.int32),
            pltpu.VMEM((E,), jnp.int32),
            pltpu.VMEM((_TPW, E), jnp.float32),
            pltpu.VMEM((_TPW, E), jnp.float32),
            pltpu.VMEM((_CH,), jnp.int32),
            pltpu.VMEM((_CH,), jnp.int32),
            pltpu.VMEM((_CH,), jnp.int32),
            pltpu.VMEM((_CH,), jnp.int32),
            pltpu.SemaphoreType.DMA,
        ],
        compiler_params=pltpu.CompilerParams(needs_layout_passes=False),
    )(_combine_body)
    return k(ysh, outs, e1a, e2a, p1a, p2a, cnta, wr0, wr1)


@jax.jit
def kernel(hidden_states, router_weight, e_score_correction_bias, up_w,
           down_w, shared_up_w, shared_down_w):
    sup = shared_up_w.astype(jnp.bfloat16)
    sdn = shared_down_w.astype(jnp.bfloat16)

    wr0, wr1, e1o, e2o, p1o, p2o, cnt = _router(
        hidden_states, router_weight, e_score_correction_bias)

    e1a = e1o.reshape(T)
    e2a = e2o.reshape(T)
    p1a = p1o.reshape(T)
    p2a = p2o.reshape(T)
    cnta = cnt.reshape(E)

    xs, witems = _sc_dispatch(hidden_states, e1a, e2a, p1a, p2a, cnta)
    ysh = _shared_mlp(hidden_states, sup, sdn)
    out_s = _gmm(witems, xs, up_w, down_w)
    return _sc_combine(ysh, out_s, e1a, e2a, p1a, p2a, cnta, wr0, wr1)


# speedup vs baseline: 1.0933x; 1.0933x over previous
import functools

import jax
import jax.numpy as jnp
from jax import lax
from jax.experimental import pallas as pl
from jax.experimental.pallas import tpu as pltpu
from jax.experimental.pallas import tpu_sc as plsc

T = 2048
H = 1024
E = 16
I = 512
IS = 1024
N_GROUP = 4
GROUP_SIZE = E // N_GROUP
SCALE = 2.5

_BT = 256
_NB = T // _BT
_NA = 2 * T
_BR = 256
_NWI = _NA // _BR + E
_NAP = (_NA // _BR + E + 1) * _BR
_PAD_TILE = _NAP // _BR - 1
_NEG = -1e30

_NC = 2
_NS = 16
_NW = _NC * _NS
_TPW = T // _NW
_CH = 16
_HF = 32


def _router_body(x_ref, rwt_ref, bias_ref,
                 wr0_ref, wr1_ref, e1_ref, e2_ref,
                 p1_ref, p2_ref, cnt_ref, counts_vmem, tri_vmem):
    blk = pl.program_id(0)
    x = x_ref[...]
    logits = jnp.dot(x, rwt_ref[...], preferred_element_type=jnp.float32)
    scores = jax.nn.sigmoid(logits)
    sfc = scores + bias_ref[...]

    cols = lax.broadcasted_iota(jnp.int32, (_BT, E), 1)
    grp = cols // GROUP_SIZE

    gscores = []
    for g in range(N_GROUP):
        vals = jnp.where(grp == g, sfc, _NEG)
        m1 = jnp.max(vals, axis=1, keepdims=True)
        i1 = jnp.min(jnp.where(vals == m1, cols, E + 1), axis=1, keepdims=True)
        vals2 = jnp.where(cols == i1, _NEG, vals)
        m2 = jnp.max(vals2, axis=1, keepdims=True)
        gscores.append(m1 + m2)

    gm1 = gscores[0]
    gi1 = jnp.zeros_like(gm1, dtype=jnp.int32)
    for g in range(1, N_GROUP):
        better = gscores[g] > gm1
        gi1 = jnp.where(better, g, gi1)
        gm1 = jnp.maximum(gscores[g], gm1)
    gm2 = jnp.full_like(gm1, _NEG)
    gi2 = jnp.zeros_like(gi1)
    for g in range(N_GROUP):
        cand = jnp.where(gi1 == g, _NEG, gscores[g])
        better = cand > gm2
        gi2 = jnp.where(better, g, gi2)
        gm2 = jnp.maximum(cand, gm2)

    group_mask = (grp == gi1) | (grp == gi2)
    msfc = jnp.where(group_mask, sfc, 0.0)

    m1 = jnp.max(msfc, axis=1, keepdims=True)
    e1 = jnp.min(jnp.where(msfc == m1, cols, E + 1), axis=1, keepdims=True)
    msfc2 = jnp.where(cols == e1, _NEG, msfc)
    m2 = jnp.max(msfc2, axis=1, keepdims=True)
    e2 = jnp.min(jnp.where(msfc2 == m2, cols, E + 1), axis=1, keepdims=True)

    sel1 = cols == e1
    sel2 = cols == e2
    w1 = jnp.sum(jnp.where(sel1, scores, 0.0), axis=1, keepdims=True)
    w2 = jnp.sum(jnp.where(sel2, scores, 0.0), axis=1, keepdims=True)
    denom = w1 + w2 + 1e-20
    w1 = w1 / denom * SCALE
    w2 = w2 / denom * SCALE

    wr0_ref[...] = jnp.broadcast_to(w1, (_BT, E))
    wr1_ref[...] = jnp.broadcast_to(w2, (_BT, E))
    e1_ref[...] = e1
    e2_ref[...] = e2

    onehot = jnp.concatenate([sel1, sel2], axis=0).astype(jnp.float32)

    @pl.when(blk == 0)
    def _():
        ri = lax.broadcasted_iota(jnp.int32, (2 * _BT, 2 * _BT), 0)
        ci = lax.broadcasted_iota(jnp.int32, (2 * _BT, 2 * _BT), 1)
        tri_vmem[...] = (ci < ri).astype(jnp.float32)

    prank = jnp.dot(tri_vmem[...], onehot,
                    preferred_element_type=jnp.float32)
    base = jnp.where(blk == 0, 0.0, counts_vmem[0:1, 0:E])
    prank = prank + base
    p1 = jnp.sum(jnp.where(sel1, prank[:_BT], 0.0), axis=1, keepdims=True)
    p2 = jnp.sum(jnp.where(sel2, prank[_BT:], 0.0), axis=1, keepdims=True)
    p1_ref[...] = p1.astype(jnp.int32)
    p2_ref[...] = p2.astype(jnp.int32)
    new_counts = base + jnp.sum(onehot, axis=0, keepdims=True)
    counts_vmem[0:1, 0:E] = new_counts
    cnt_ref[...] = new_counts.astype(jnp.int32)


def _router(hidden_states, router_weight, bias):
    return pl.pallas_call(
        _router_body,
        grid=(_NB,),
        in_specs=[
            pl.BlockSpec((_BT, H), lambda i: (i, 0)),
            pl.BlockSpec((H, E), lambda i: (0, 0)),
            pl.BlockSpec((1, E), lambda i: (0, 0)),
        ],
        out_specs=[
            pl.BlockSpec((_BT, E), lambda i: (i, 0)),
            pl.BlockSpec((_BT, E), lambda i: (i, 0)),
            pl.BlockSpec((_BT, 1), lambda i: (i, 0)),
            pl.BlockSpec((_BT, 1), lambda i: (i, 0)),
            pl.BlockSpec((_BT, 1), lambda i: (i, 0)),
            pl.BlockSpec((_BT, 1), lambda i: (i, 0)),
            pl.BlockSpec((1, E), lambda i: (0, 0)),
        ],
        out_shape=[
            jax.ShapeDtypeStruct((T, E), jnp.float32),
            jax.ShapeDtypeStruct((T, E), jnp.float32),
            jax.ShapeDtypeStruct((T, 1), jnp.int32),
            jax.ShapeDtypeStruct((T, 1), jnp.int32),
            jax.ShapeDtypeStruct((T, 1), jnp.int32),
            jax.ShapeDtypeStruct((T, 1), jnp.int32),
            jax.ShapeDtypeStruct((1, E), jnp.int32),
        ],
        scratch_shapes=[pltpu.VMEM((8, 128), jnp.float32),
                        pltpu.VMEM((2 * _BT, 2 * _BT), jnp.float32)],
        compiler_params=pltpu.CompilerParams(
            dimension_semantics=("arbitrary",)),
    )(hidden_states, router_weight.T, bias.reshape(1, E))


def _shared_body(x_ref, sup_ref, sdn_ref, ysh_ref):
    xb = x_ref[...].astype(jnp.bfloat16)
    h = jnp.dot(xb, sup_ref[...], preferred_element_type=jnp.float32)
    r = jnp.maximum(h, 0.0)
    rr = (r * r).astype(jnp.bfloat16)
    ysh_ref[...] = jnp.dot(rr, sdn_ref[...], preferred_element_type=jnp.float32)


def _shared_mlp(hidden_states, sup, sdn):
    return pl.pallas_call(
        _shared_body,
        grid=(_NB,),
        in_specs=[
            pl.BlockSpec((_BT, H), lambda i: (i, 0)),
            pl.BlockSpec((H, IS), lambda i: (0, 0)),
            pl.BlockSpec((IS, H), lambda i: (0, 0)),
        ],
        out_specs=pl.BlockSpec((_BT, H), lambda i: (i, 0)),
        out_shape=jax.ShapeDtypeStruct((T, H), jnp.float32),
    )(hidden_states, sup, sdn)


def _poff_from_counts(cnt_v):
    pcnt = ((cnt_v + (_BR - 1)) // _BR) * _BR
    return plsc.cumsum(pcnt) - pcnt, pcnt


def _lane_select(vec, e):
    lidx = lax.broadcasted_iota(jnp.int32, (E,), 0)
    return jnp.sum(jnp.where(lidx == e, vec, 0))


def _sc_mesh():
    return plsc.VectorSubcoreMesh(core_axis_name="c", subcore_axis_name="s")


def _dispatch_body(hid, e1a, e2a, p1a, p2a, cnta, xs, wout,
                   rows_v, e1_v, e2_v, p1_v, p2_v, cnt_v, poff_v, wit_v, sem):
    wid = lax.axis_index("s") * _NC + lax.axis_index("c")
    base = wid * _TPW
    pltpu.sync_copy(cnta, cnt_v)
    pltpu.sync_copy(hid.at[pl.ds(base, _TPW)], rows_v)
    pltpu.sync_copy(e1a.at[pl.ds(base, _TPW)], e1_v)
    pltpu.sync_copy(e2a.at[pl.ds(base, _TPW)], e2_v)
    pltpu.sync_copy(p1a.at[pl.ds(base, _TPW)], p1_v)
    pltpu.sync_copy(p2a.at[pl.ds(base, _TPW)], p2_v)

    cnt = cnt_v[...]
    poff, _ = _poff_from_counts(cnt)
    poff_v[...] = poff

    for ch in range(_TPW // _CH):
        sl = pl.ds(ch * _CH, _CH)
        p1_v[sl] = plsc.load_gather(poff_v, [e1_v[sl]]) + p1_v[sl]
        p2_v[sl] = plsc.load_gather(poff_v, [e2_v[sl]]) + p2_v[sl]
    cps = [pltpu.async_copy(rows_v, xs.at[p1_v], sem),
           pltpu.async_copy(rows_v, xs.at[p2_v], sem)]

    ntile = (cnt + (_BR - 1)) // _BR
    tbsum = plsc.cumsum(ntile)
    tb = tbsum - ntile
    total = jnp.max(tbsum)
    for half in range(_NWI // E):
        cvec = lax.broadcasted_iota(jnp.int32, (E,), 0) + half * E
        e_of = jnp.zeros((E,), jnp.int32)
        for e in range(E):
            e_of = e_of + (jnp.where(_lane_select(tb, e) <= cvec, 1, 0))
        e_of = e_of - 1
        jv = cvec - _gather_reg(tb, e_of, poff_v)
        st = _gather_reg(poff, e_of, poff_v) + jv * _BR
        seg_end = (_gather_reg(poff, e_of, poff_v)
                   + _gather_reg(cnt, e_of, poff_v))
        en = jnp.minimum(st + _BR, seg_end)
        msk = cvec < total
        sl = pl.ds(half * E, E)
        wit_v[0, sl] = jnp.where(msk, e_of, 0)
        wit_v[1, sl] = jnp.where(msk, st // _BR, _PAD_TILE)
        wit_v[2, sl] = jnp.where(msk, en, 0)
        wit_v[3, sl] = jnp.zeros((E,), jnp.int32)

    @pl.when(wid == 0)
    def _():
        pltpu.sync_copy(wit_v, wout)

    for cp in cps:
        cp.wait()


def _gather_reg(vec, idx, scratch_v):
    scratch_v[...] = vec
    return plsc.load_gather(scratch_v, [idx])


def _sc_dispatch(hidden_states, e1a, e2a, p1a, p2a, cnta):
    k = functools.partial(
        pl.kernel,
        mesh=_sc_mesh(),
        out_type=[
            jax.ShapeDtypeStruct((_NAP, H), jnp.float32),
            jax.ShapeDtypeStruct((4, _NWI), jnp.int32),
        ],
        scratch_types=[
            pltpu.VMEM((_TPW, H), jnp.float32),
            pltpu.VMEM((_TPW,), jnp.int32),
            pltpu.VMEM((_TPW,), jnp.int32),
            pltpu.VMEM((_TPW,), jnp.int32),
            pltpu.VMEM((_TPW,), jnp.int32),
            pltpu.VMEM((E,), jnp.int32),
            pltpu.VMEM((E,), jnp.int32),
            pltpu.VMEM((4, _NWI), jnp.int32),
            pltpu.SemaphoreType.DMA,
        ],
        compiler_params=pltpu.CompilerParams(needs_layout_passes=False),
    )(_dispatch_body)
    return k(hidden_states, e1a, e2a, p1a, p2a, cnta)


def _gmm_body(wit_ref, xs_ref, up_ref, dn_ref, out_ref):
    wi = pl.program_id(0)
    tile = wit_ref[1, wi]
    end = wit_ref[2, wi]

    @pl.when(end > 0)
    def _():
        x = xs_ref[...].astype(jnp.bfloat16)
        up = up_ref[0].astype(jnp.bfloat16)
        dn = dn_ref[0].astype(jnp.bfloat16)
        h = jnp.dot(x, up, preferred_element_type=jnp.float32)
        r = jnp.maximum(h, 0.0)
        rr = (r * r).astype(jnp.bfloat16)
        yc = jnp.dot(rr, dn, preferred_element_type=jnp.float32)
        rows = tile * _BR + lax.broadcasted_iota(jnp.int32, (_BR, 1), 0)
        out_ref[...] = jnp.where(rows < end, yc, 0.0)


def _gmm(witems, xs, up_w, down_w):
    return pl.pallas_call(
        _gmm_body,
        grid_spec=pltpu.PrefetchScalarGridSpec(
            num_scalar_prefetch=1,
            grid=(_NWI,),
            in_specs=[
                pl.BlockSpec((_BR, H), lambda wi, wa: (wa[1, wi], 0)),
                pl.BlockSpec((1, H, I), lambda wi, wa: (wa[0, wi], 0, 0)),
                pl.BlockSpec((1, I, H), lambda wi, wa: (wa[0, wi], 0, 0)),
            ],
            out_specs=pl.BlockSpec((_BR, H), lambda wi, wa: (wa[1, wi], 0)),
        ),
        out_shape=jax.ShapeDtypeStruct((_NAP, H), jnp.float32),
        compiler_params=pltpu.CompilerParams(
            dimension_semantics=("arbitrary",)),
    )(witems, xs, up_w, down_w)


def _combine_body(ysh, outs, e1a, e2a, p1a, p2a, cnta, wr0, wr1, y,
                  acc_v, r0_v, r1_v, r0b_v, r1b_v,
                  e1_v, e2_v, p1_v, p2_v, cnt_v, poff_v, w0_v, w1_v,
                  i0a_v, i1a_v, i0b_v, i1b_v, sem):
    wid = lax.axis_index("s") * _NC + lax.axis_index("c")
    base = wid * _TPW
    pltpu.sync_copy(cnta, cnt_v)
    pltpu.sync_copy(e1a.at[pl.ds(base, _TPW)], e1_v)
    pltpu.sync_copy(e2a.at[pl.ds(base, _TPW)], e2_v)
    pltpu.sync_copy(p1a.at[pl.ds(base, _TPW)], p1_v)
    pltpu.sync_copy(p2a.at[pl.ds(base, _TPW)], p2_v)
    pltpu.sync_copy(wr0.at[pl.ds(base, _TPW)], w0_v)
    pltpu.sync_copy(wr1.at[pl.ds(base, _TPW)], w1_v)
    poff, _ = _poff_from_counts(cnt_v[...])
    poff_v[...] = poff
    for ch in range(_TPW // _CH):
        sl = pl.ds(ch * _CH, _CH)
        p1_v[sl] = plsc.load_gather(poff_v, [e1_v[sl]]) + p1_v[sl]
        p2_v[sl] = plsc.load_gather(poff_v, [e2_v[sl]]) + p2_v[sl]

    nch = _TPW // _CH
    bufs = [(r0_v, r1_v), (r0b_v, r1b_v)]
    ibufs = [(i0a_v, i1a_v), (i0b_v, i1b_v)]

    def fire(c):
        sl = pl.ds(c * _CH, _CH)
        r0, r1 = bufs[c % 2]
        ia, ib = ibufs[c % 2]
        ia[...] = p1_v[sl]
        ib[...] = p2_v[sl]
        return (pltpu.async_copy(outs.at[ia], r0, sem),
                pltpu.async_copy(outs.at[ib], r1, sem))

    chunks_per_half = _HF // _CH
    cps = fire(0)
    for c in range(nch):
        hf, cl = divmod(c, chunks_per_half)
        if cl == 0:
            pltpu.sync_copy(ysh.at[pl.ds(base + hf * _HF, _HF)], acc_v)
        cps[0].wait()
        cps[1].wait()
        if c + 1 < nch:
            nxt = fire(c + 1)
        r0, r1 = bufs[c % 2]
        for r in range(_CH):
            row = c * _CH + r
            arow = cl * _CH + r
            wv0 = w0_v[row, :]
            wv1 = w1_v[row, :]

            @plsc.parallel_loop(0, H, step=16, unroll=8)
            def _(off, arow=arow, r=r, wv0=wv0, wv1=wv1, r0=r0, r1=r1):
                sl = pl.ds(off, 16)
                acc_v[arow, sl] = (acc_v[arow, sl] + wv0 * r0[r, sl]
                                   + wv1 * r1[r, sl])

        if cl == chunks_per_half - 1:
            pltpu.sync_copy(acc_v, y.at[pl.ds(base + hf * _HF, _HF)])
        if c + 1 < nch:
            cps = nxt


def _sc_combine(ysh, outs, e1a, e2a, p1a, p2a, cnta, wr0, wr1):
    k = functools.partial(
        pl.kernel,
        mesh=_sc_mesh(),
        out_type=jax.ShapeDtypeStruct((T, H), jnp.float32),
        scratch_types=[
            pltpu.VMEM((_HF, H), jnp.float32),
            pltpu.VMEM((_CH, H), jnp.float32),
            pltpu.VMEM((_CH, H), jnp.float32),
            pltpu.VMEM((_CH, H), jnp.float32),
            pltpu.VMEM((_CH, H), jnp.float32),
            pltpu.VMEM((_TPW,), jnp.int32),
            pltpu.VMEM((_TPW,), jnp.int32),
            pltpu.VMEM((_TPW,), jnp.int32),
            pltpu.VMEM((_TPW,), jnp.int32),
            pltpu.VMEM((E,), jnp.int32),
            pltpu.VMEM((E,), jnp.int32),
            pltpu.VMEM((_TPW, E), jnp.float32),
            pltpu.VMEM((_TPW, E), jnp.float32),
            pltpu.VMEM((_CH,), jnp.int32),
            pltpu.VMEM((_CH,), jnp.int32),
            pltpu.VMEM((_CH,), jnp.int32),
            pltpu.VMEM((_CH,), jnp.int32),
            pltpu.SemaphoreType.DMA,
        ],
        compiler_params=pltpu.CompilerParams(needs_layout_passes=False),
    )(_combine_body)
    return k(ysh, outs, e1a, e2a, p1a, p2a, cnta, wr0, wr1)


@jax.jit
def kernel(hidden_states, router_weight, e_score_correction_bias, up_w,
           down_w, shared_up_w, shared_down_w):
    sup = shared_up_w.astype(jnp.bfloat16)
    sdn = shared_down_w.astype(jnp.bfloat16)

    wr0, wr1, e1o, e2o, p1o, p2o, cnt = _router(
        hidden_states, router_weight, e_score_correction_bias)

    e1a = e1o.reshape(T)
    e2a = e2o.reshape(T)
    p1a = p1o.reshape(T)
    p2a = p2o.reshape(T)
    cnta = cnt.reshape(E)

    xs, witems = _sc_dispatch(hidden_states, e1a, e2a, p1a, p2a, cnta)
    ysh = _shared_mlp(hidden_states, sup, sdn)
    out_s = _gmm(witems, xs, up_w, down_w)
    return _sc_combine(ysh, out_s, e1a, e2a, p1a, p2a, cnta, wr0, wr1)


# speedup vs baseline: 1.1080x; 1.0134x over previous
import functools

import jax
import jax.numpy as jnp
from jax import lax
from jax.experimental import pallas as pl
from jax.experimental.pallas import tpu as pltpu
from jax.experimental.pallas import tpu_sc as plsc

T = 2048
H = 1024
E = 16
I = 512
IS = 1024
N_GROUP = 4
GROUP_SIZE = E // N_GROUP
SCALE = 2.5

_BT = 256
_NB = T // _BT
_NA = 2 * T
_BR = 256
_NWI = _NA // _BR + E
_NAP = (_NA // _BR + E + 1) * _BR
_PAD_TILE = _NAP // _BR - 1
_NEG = -1e30

_NC = 2
_NS = 16
_NW = _NC * _NS
_TPW = T // _NW
_CH = 16
_HF = 32


def _router_body(x_ref, rwt_ref, bias_ref,
                 wr0_ref, wr1_ref, e1_ref, e2_ref,
                 p1_ref, p2_ref, cnt_ref, counts_vmem, tri_vmem):
    blk = pl.program_id(0)
    x = x_ref[...]
    logits = jnp.dot(x, rwt_ref[...], preferred_element_type=jnp.float32)
    scores = jax.nn.sigmoid(logits)
    sfc = scores + bias_ref[...]

    cols = lax.broadcasted_iota(jnp.int32, (_BT, E), 1)
    grp = cols // GROUP_SIZE

    gscores = []
    for g in range(N_GROUP):
        vals = jnp.where(grp == g, sfc, _NEG)
        m1 = jnp.max(vals, axis=1, keepdims=True)
        i1 = jnp.min(jnp.where(vals == m1, cols, E + 1), axis=1, keepdims=True)
        vals2 = jnp.where(cols == i1, _NEG, vals)
        m2 = jnp.max(vals2, axis=1, keepdims=True)
        gscores.append(m1 + m2)

    gm1 = gscores[0]
    gi1 = jnp.zeros_like(gm1, dtype=jnp.int32)
    for g in range(1, N_GROUP):
        better = gscores[g] > gm1
        gi1 = jnp.where(better, g, gi1)
        gm1 = jnp.maximum(gscores[g], gm1)
    gm2 = jnp.full_like(gm1, _NEG)
    gi2 = jnp.zeros_like(gi1)
    for g in range(N_GROUP):
        cand = jnp.where(gi1 == g, _NEG, gscores[g])
        better = cand > gm2
        gi2 = jnp.where(better, g, gi2)
        gm2 = jnp.maximum(cand, gm2)

    group_mask = (grp == gi1) | (grp == gi2)
    msfc = jnp.where(group_mask, sfc, 0.0)

    m1 = jnp.max(msfc, axis=1, keepdims=True)
    e1 = jnp.min(jnp.where(msfc == m1, cols, E + 1), axis=1, keepdims=True)
    msfc2 = jnp.where(cols == e1, _NEG, msfc)
    m2 = jnp.max(msfc2, axis=1, keepdims=True)
    e2 = jnp.min(jnp.where(msfc2 == m2, cols, E + 1), axis=1, keepdims=True)

    sel1 = cols == e1
    sel2 = cols == e2
    w1 = jnp.sum(jnp.where(sel1, scores, 0.0), axis=1, keepdims=True)
    w2 = jnp.sum(jnp.where(sel2, scores, 0.0), axis=1, keepdims=True)
    denom = w1 + w2 + 1e-20
    w1 = w1 / denom * SCALE
    w2 = w2 / denom * SCALE

    wr0_ref[...] = jnp.broadcast_to(w1, (_BT, E))
    wr1_ref[...] = jnp.broadcast_to(w2, (_BT, E))
    e1_ref[...] = e1
    e2_ref[...] = e2

    onehot = jnp.concatenate([sel1, sel2], axis=0).astype(jnp.float32)

    @pl.when(blk == 0)
    def _():
        ri = lax.broadcasted_iota(jnp.int32, (2 * _BT, 2 * _BT), 0)
        ci = lax.broadcasted_iota(jnp.int32, (2 * _BT, 2 * _BT), 1)
        tri_vmem[...] = (ci < ri).astype(jnp.float32)

    prank = jnp.dot(tri_vmem[...], onehot,
                    preferred_element_type=jnp.float32)
    base = jnp.where(blk == 0, 0.0, counts_vmem[0:1, 0:E])
    prank = prank + base
    p1 = jnp.sum(jnp.where(sel1, prank[:_BT], 0.0), axis=1, keepdims=True)
    p2 = jnp.sum(jnp.where(sel2, prank[_BT:], 0.0), axis=1, keepdims=True)
    p1_ref[...] = p1.astype(jnp.int32)
    p2_ref[...] = p2.astype(jnp.int32)
    new_counts = base + jnp.sum(onehot, axis=0, keepdims=True)
    counts_vmem[0:1, 0:E] = new_counts
    cnt_ref[...] = new_counts.astype(jnp.int32)


def _router(hidden_states, router_weight, bias):
    return pl.pallas_call(
        _router_body,
        grid=(_NB,),
        in_specs=[
            pl.BlockSpec((_BT, H), lambda i: (i, 0)),
            pl.BlockSpec((H, E), lambda i: (0, 0)),
            pl.BlockSpec((1, E), lambda i: (0, 0)),
        ],
        out_specs=[
            pl.BlockSpec((_BT, E), lambda i: (i, 0)),
            pl.BlockSpec((_BT, E), lambda i: (i, 0)),
            pl.BlockSpec((_BT, 1), lambda i: (i, 0)),
            pl.BlockSpec((_BT, 1), lambda i: (i, 0)),
            pl.BlockSpec((_BT, 1), lambda i: (i, 0)),
            pl.BlockSpec((_BT, 1), lambda i: (i, 0)),
            pl.BlockSpec((1, E), lambda i: (0, 0)),
        ],
        out_shape=[
            jax.ShapeDtypeStruct((T, E), jnp.float32),
            jax.ShapeDtypeStruct((T, E), jnp.float32),
            jax.ShapeDtypeStruct((T, 1), jnp.int32),
            jax.ShapeDtypeStruct((T, 1), jnp.int32),
            jax.ShapeDtypeStruct((T, 1), jnp.int32),
            jax.ShapeDtypeStruct((T, 1), jnp.int32),
            jax.ShapeDtypeStruct((1, E), jnp.int32),
        ],
        scratch_shapes=[pltpu.VMEM((8, 128), jnp.float32),
                        pltpu.VMEM((2 * _BT, 2 * _BT), jnp.float32)],
        compiler_params=pltpu.CompilerParams(
            dimension_semantics=("arbitrary",)),
    )(hidden_states, router_weight.T, bias.reshape(1, E))


def _shared_body(x_ref, sup_ref, sdn_ref, ysh_ref):
    xb = x_ref[...].astype(jnp.bfloat16)
    h = jnp.dot(xb, sup_ref[...], preferred_element_type=jnp.float32)
    r = jnp.maximum(h, 0.0)
    rr = (r * r).astype(jnp.bfloat16)
    ysh_ref[...] = jnp.dot(rr, sdn_ref[...], preferred_element_type=jnp.float32)


def _shared_mlp(hidden_states, sup, sdn):
    return pl.pallas_call(
        _shared_body,
        grid=(_NB,),
        in_specs=[
            pl.BlockSpec((_BT, H), lambda i: (i, 0)),
            pl.BlockSpec((H, IS), lambda i: (0, 0)),
            pl.BlockSpec((IS, H), lambda i: (0, 0)),
        ],
        out_specs=pl.BlockSpec((_BT, H), lambda i: (i, 0)),
        out_shape=jax.ShapeDtypeStruct((T, H), jnp.float32),
    )(hidden_states, sup, sdn)


def _poff_from_counts(cnt_v):
    pcnt = ((cnt_v + (_BR - 1)) // _BR) * _BR
    return plsc.cumsum(pcnt) - pcnt, pcnt


def _lane_select(vec, e):
    lidx = lax.broadcasted_iota(jnp.int32, (E,), 0)
    return jnp.sum(jnp.where(lidx == e, vec, 0))


def _sc_mesh():
    return plsc.VectorSubcoreMesh(core_axis_name="c", subcore_axis_name="s")


def _dispatch_body(hid, e1a, e2a, p1a, p2a, cnta, xs, wout,
                   rows_v, e1_v, e2_v, p1_v, p2_v, cnt_v, poff_v, wit_v, sem):
    wid = lax.axis_index("s") * _NC + lax.axis_index("c")
    base = wid * _TPW
    pltpu.sync_copy(cnta, cnt_v)
    pltpu.sync_copy(hid.at[pl.ds(base, _TPW)], rows_v)
    pltpu.sync_copy(e1a.at[pl.ds(base, _TPW)], e1_v)
    pltpu.sync_copy(e2a.at[pl.ds(base, _TPW)], e2_v)
    pltpu.sync_copy(p1a.at[pl.ds(base, _TPW)], p1_v)
    pltpu.sync_copy(p2a.at[pl.ds(base, _TPW)], p2_v)

    cnt = cnt_v[...]
    poff, _ = _poff_from_counts(cnt)
    poff_v[...] = poff

    for ch in range(_TPW // _CH):
        sl = pl.ds(ch * _CH, _CH)
        p1_v[sl] = plsc.load_gather(poff_v, [e1_v[sl]]) + p1_v[sl]
        p2_v[sl] = plsc.load_gather(poff_v, [e2_v[sl]]) + p2_v[sl]
    cps = [pltpu.async_copy(rows_v, xs.at[p1_v], sem),
           pltpu.async_copy(rows_v, xs.at[p2_v], sem)]

    ntile = (cnt + (_BR - 1)) // _BR
    tbsum = plsc.cumsum(ntile)
    tb = tbsum - ntile
    total = jnp.max(tbsum)
    for half in range(_NWI // E):
        cvec = lax.broadcasted_iota(jnp.int32, (E,), 0) + half * E
        e_of = jnp.zeros((E,), jnp.int32)
        for e in range(E):
            e_of = e_of + (jnp.where(_lane_select(tb, e) <= cvec, 1, 0))
        e_of = e_of - 1
        jv = cvec - _gather_reg(tb, e_of, poff_v)
        st = _gather_reg(poff, e_of, poff_v) + jv * _BR
        seg_end = (_gather_reg(poff, e_of, poff_v)
                   + _gather_reg(cnt, e_of, poff_v))
        en = jnp.minimum(st + _BR, seg_end)
        msk = cvec < total
        sl = pl.ds(half * E, E)
        wit_v[0, sl] = e_of
        wit_v[1, sl] = jnp.where(msk, st // _BR, _PAD_TILE)
        wit_v[2, sl] = jnp.where(msk, en, 0)
        wit_v[3, sl] = jnp.zeros((E,), jnp.int32)

    @pl.when(wid == 0)
    def _():
        pltpu.sync_copy(wit_v, wout)

    for cp in cps:
        cp.wait()


def _gather_reg(vec, idx, scratch_v):
    scratch_v[...] = vec
    return plsc.load_gather(scratch_v, [idx])


def _sc_dispatch(hidden_states, e1a, e2a, p1a, p2a, cnta):
    k = functools.partial(
        pl.kernel,
        mesh=_sc_mesh(),
        out_type=[
            jax.ShapeDtypeStruct((_NAP, H), jnp.float32),
            jax.ShapeDtypeStruct((4, _NWI), jnp.int32),
        ],
        scratch_types=[
            pltpu.VMEM((_TPW, H), jnp.float32),
            pltpu.VMEM((_TPW,), jnp.int32),
            pltpu.VMEM((_TPW,), jnp.int32),
            pltpu.VMEM((_TPW,), jnp.int32),
            pltpu.VMEM((_TPW,), jnp.int32),
            pltpu.VMEM((E,), jnp.int32),
            pltpu.VMEM((E,), jnp.int32),
            pltpu.VMEM((4, _NWI), jnp.int32),
            pltpu.SemaphoreType.DMA,
        ],
        compiler_params=pltpu.CompilerParams(needs_layout_passes=False),
    )(_dispatch_body)
    return k(hidden_states, e1a, e2a, p1a, p2a, cnta)


def _gmm_body(wit_ref, xs_ref, up_ref, dn_ref, out_ref):
    wi = pl.program_id(0)
    tile = wit_ref[1, wi]
    end = wit_ref[2, wi]

    @pl.when(end > 0)
    def _():
        x = xs_ref[...].astype(jnp.bfloat16)
        up = up_ref[0].astype(jnp.bfloat16)
        dn = dn_ref[0].astype(jnp.bfloat16)
        h = jnp.dot(x, up, preferred_element_type=jnp.float32)
        r = jnp.maximum(h, 0.0)
        rr = (r * r).astype(jnp.bfloat16)
        yc = jnp.dot(rr, dn, preferred_element_type=jnp.float32)
        rows = tile * _BR + lax.broadcasted_iota(jnp.int32, (_BR, 1), 0)
        out_ref[...] = jnp.where(rows < end, yc, 0.0)


def _gmm(witems, xs, up_w, down_w):
    return pl.pallas_call(
        _gmm_body,
        grid_spec=pltpu.PrefetchScalarGridSpec(
            num_scalar_prefetch=1,
            grid=(_NWI,),
            in_specs=[
                pl.BlockSpec((_BR, H), lambda wi, wa: (wa[1, wi], 0)),
                pl.BlockSpec((1, H, I), lambda wi, wa: (wa[0, wi], 0, 0)),
                pl.BlockSpec((1, I, H), lambda wi, wa: (wa[0, wi], 0, 0)),
            ],
            out_specs=pl.BlockSpec((_BR, H), lambda wi, wa: (wa[1, wi], 0)),
        ),
        out_shape=jax.ShapeDtypeStruct((_NAP, H), jnp.float32),
        compiler_params=pltpu.CompilerParams(
            dimension_semantics=("arbitrary",)),
    )(witems, xs, up_w, down_w)


def _combine_body(ysh, outs, e1a, e2a, p1a, p2a, cnta, wr0, wr1, y,
                  acc_v, r0_v, r1_v, r0b_v, r1b_v,
                  e1_v, e2_v, p1_v, p2_v, cnt_v, poff_v, w0_v, w1_v,
                  i0a_v, i1a_v, i0b_v, i1b_v, sem):
    wid = lax.axis_index("s") * _NC + lax.axis_index("c")
    base = wid * _TPW
    pltpu.sync_copy(cnta, cnt_v)
    pltpu.sync_copy(e1a.at[pl.ds(base, _TPW)], e1_v)
    pltpu.sync_copy(e2a.at[pl.ds(base, _TPW)], e2_v)
    pltpu.sync_copy(p1a.at[pl.ds(base, _TPW)], p1_v)
    pltpu.sync_copy(p2a.at[pl.ds(base, _TPW)], p2_v)
    pltpu.sync_copy(wr0.at[pl.ds(base, _TPW)], w0_v)
    pltpu.sync_copy(wr1.at[pl.ds(base, _TPW)], w1_v)
    poff, _ = _poff_from_counts(cnt_v[...])
    poff_v[...] = poff
    for ch in range(_TPW // _CH):
        sl = pl.ds(ch * _CH, _CH)
        p1_v[sl] = plsc.load_gather(poff_v, [e1_v[sl]]) + p1_v[sl]
        p2_v[sl] = plsc.load_gather(poff_v, [e2_v[sl]]) + p2_v[sl]

    nch = _TPW // _CH
    bufs = [(r0_v, r1_v), (r0b_v, r1b_v)]
    ibufs = [(i0a_v, i1a_v), (i0b_v, i1b_v)]

    def fire(c):
        sl = pl.ds(c * _CH, _CH)
        r0, r1 = bufs[c % 2]
        ia, ib = ibufs[c % 2]
        ia[...] = p1_v[sl]
        ib[...] = p2_v[sl]
        return (pltpu.async_copy(outs.at[ia], r0, sem),
                pltpu.async_copy(outs.at[ib], r1, sem))

    chunks_per_half = _HF // _CH
    cps = fire(0)
    for c in range(nch):
        hf, cl = divmod(c, chunks_per_half)
        if cl == 0:
            pltpu.sync_copy(ysh.at[pl.ds(base + hf * _HF, _HF)], acc_v)
        cps[0].wait()
        cps[1].wait()
        if c + 1 < nch:
            nxt = fire(c + 1)
        r0, r1 = bufs[c % 2]
        for r in range(_CH):
            row = c * _CH + r
            arow = cl * _CH + r
            wv0 = w0_v[row, :]
            wv1 = w1_v[row, :]

            @plsc.parallel_loop(0, H, step=16, unroll=8)
            def _(off, arow=arow, r=r, wv0=wv0, wv1=wv1, r0=r0, r1=r1):
                sl = pl.ds(off, 16)
                acc_v[arow, sl] = (acc_v[arow, sl] + wv0 * r0[r, sl]
                                   + wv1 * r1[r, sl])

        if cl == chunks_per_half - 1:
            pltpu.sync_copy(acc_v, y.at[pl.ds(base + hf * _HF, _HF)])
        if c + 1 < nch:
            cps = nxt


def _sc_combine(ysh, outs, e1a, e2a, p1a, p2a, cnta, wr0, wr1):
    k = functools.partial(
        pl.kernel,
        mesh=_sc_mesh(),
        out_type=jax.ShapeDtypeStruct((T, H), jnp.float32),
        scratch_types=[
            pltpu.VMEM((_HF, H), jnp.float32),
            pltpu.VMEM((_CH, H), jnp.float32),
            pltpu.VMEM((_CH, H), jnp.float32),
            pltpu.VMEM((_CH, H), jnp.float32),
            pltpu.VMEM((_CH, H), jnp.float32),
            pltpu.VMEM((_TPW,), jnp.int32),
            pltpu.VMEM((_TPW,), jnp.int32),
            pltpu.VMEM((_TPW,), jnp.int32),
            pltpu.VMEM((_TPW,), jnp.int32),
            pltpu.VMEM((E,), jnp.int32),
            pltpu.VMEM((E,), jnp.int32),
            pltpu.VMEM((_TPW, E), jnp.float32),
            pltpu.VMEM((_TPW, E), jnp.float32),
            pltpu.VMEM((_CH,), jnp.int32),
            pltpu.VMEM((_CH,), jnp.int32),
            pltpu.VMEM((_CH,), jnp.int32),
            pltpu.VMEM((_CH,), jnp.int32),
            pltpu.SemaphoreType.DMA,
        ],
        compiler_params=pltpu.CompilerParams(needs_layout_passes=False),
    )(_combine_body)
    return k(ysh, outs, e1a, e2a, p1a, p2a, cnta, wr0, wr1)


@jax.jit
def kernel(hidden_states, router_weight, e_score_correction_bias, up_w,
           down_w, shared_up_w, shared_down_w):
    sup = shared_up_w.astype(jnp.bfloat16)
    sdn = shared_down_w.astype(jnp.bfloat16)

    wr0, wr1, e1o, e2o, p1o, p2o, cnt = _router(
        hidden_states, router_weight, e_score_correction_bias)

    e1a = e1o.reshape(T)
    e2a = e2o.reshape(T)
    p1a = p1o.reshape(T)
    p2a = p2o.reshape(T)
    cnta = cnt.reshape(E)

    xs, witems = _sc_dispatch(hidden_states, e1a, e2a, p1a, p2a, cnta)
    ysh = _shared_mlp(hidden_states, sup, sdn)
    out_s = _gmm(witems, xs, up_w, down_w)
    return _sc_combine(ysh, out_s, e1a, e2a, p1a, p2a, cnta, wr0, wr1)
